# Initial kernel scaffold; baseline (speedup 1.0000x reference)
import jax
import jax.numpy as jnp
from jax.experimental import pallas as pl


def kernel(x, y, z):
    raise NotImplementedError("write your pallas kernel here")


# speedup vs baseline: 1.0252x; 1.0252x over previous
import functools

import jax
import jax.numpy as jnp
from jax import lax
from jax.experimental import pallas as pl
from jax.experimental.pallas import tpu as pltpu
from jax.experimental.pallas import tpu_sc as plsc

_LANES = 16
_NC, _NS = 2, 16
_NW = _NC * _NS
_ROWS = 64 * 128
_B = 8
_GROUPS = _ROWS // _B
_GPW = _GROUPS // _NW


def _fold_spec(oh, ow, kh, kw, sh, sw, ph, pw, dh, dw):
    assert sw == 1, "all three folds have unit output-width stride"
    Lh = (oh + 2 * ph - dh * (kh - 1) - 1) // sh + 1
    Lw = (ow + 2 * pw - dw * (kw - 1) - 1) // sw + 1
    slab = kh * kw * Lh * Lw
    olen = oh * ow
    ops = []
    for ki in range(kh):
        for kj in range(kw):
            for lh in range(Lh):
                r = lh * sh + ki * dh - ph
                if r < 0 or r >= oh:
                    continue
                c0 = kj * dw - pw
                lw0 = max(0, -c0)
                lw1 = min(Lw, ow - c0)
                if lw1 <= lw0:
                    continue
                src = ((ki * kw + kj) * Lh + lh) * Lw + lw0
                dst = r * ow + lw0 + c0
                ln = lw1 - lw0
                off = 0
                while off < ln:
                    ops.append((src + off, dst + off, min(_LANES, ln - off)))
                    off += _LANES
    return dict(slab=slab, olen=olen, ops=ops)


_SPECS = (
    _fold_spec(22, 22, 3, 3, 1, 1, 0, 0, 1, 1),
    _fold_spec(17, 18, 2, 4, 2, 1, 2, 2, 1, 1),
    _fold_spec(5, 11, 2, 3, 1, 1, 2, 4, 1, 2),
)
_PAD = 2 * _LANES


def _sc_fold_kernel(xh, yh, zh, oxh, oyh, ozh,
                    ix0, iy0, iz0, ix1, iy1, iz1,
                    ox0, oy0, oz0, ox1, oy1, oz1,
                    si0, si1, so0, so1):
    wid = lax.axis_index("s") * _NC + lax.axis_index("c")
    g0 = wid * _GPW
    ins = ((xh, yh, zh))
    outs = ((oxh, oyh, ozh))
    in_slots = ((ix0, iy0, iz0), (ix1, iy1, iz1))
    out_slots = ((ox0, oy0, oz0), (ox1, oy1, oz1))
    in_sems = (si0, si1)
    out_sems = (so0, so1)

    zero = jnp.zeros((_LANES,), jnp.float32)
    iota = lax.iota(jnp.int32, _LANES)
    masks = {ln: iota < ln
             for spec in _SPECS for (_, _, ln) in spec["ops"] if ln < _LANES}

    def in_copy(g, b):
        for hbm, buf, spec in zip(ins, in_slots[b], _SPECS):
            sz = _B * spec["slab"]
            yield pltpu.make_async_copy(
                hbm.at[pl.ds(g * sz, sz)], buf.at[pl.ds(0, sz)], in_sems[b])

    def out_copy(g, b):
        for hbm, buf, spec in zip(outs, out_slots[b], _SPECS):
            sz = _B * spec["olen"]
            yield pltpu.make_async_copy(
                buf.at[pl.ds(0, sz)], hbm.at[pl.ds(g * sz, sz)], out_sems[b])

    def compute_group(b):
        bufs_i = in_slots[b]
        bufs_o = out_slots[b]

        @pl.loop(0, _B)
        def _(i):
            for buf_i, buf_o, spec in zip(bufs_i, bufs_o, _SPECS):
                bi = i * spec["slab"]
                bo = i * spec["olen"]
                for off in range(0, spec["olen"], _LANES):
                    buf_o[pl.ds(bo + off, _LANES)] = zero
                for so, do, ln in spec["ops"]:
                    v = buf_i[pl.ds(bi + so, _LANES)]
                    if ln < _LANES:
                        v = jnp.where(masks[ln], v, 0.0)
                    plsc.addupdate(buf_o.at[pl.ds(bo + do, _LANES)], v)

    for c in in_copy(g0, 0):
        c.start()
    for c in in_copy(g0 + 1, 1):
        c.start()

    @pl.loop(0, _GPW, step=2)
    def _(t):
        for b in (0, 1):
            g = g0 + t + b
            for c in in_copy(g, b):
                c.wait()

            @pl.when(t + b >= 2)
            def _():
                for c in out_copy(g - 2, b):
                    c.wait()

            compute_group(b)
            for c in out_copy(g, b):
                c.start()

            @pl.when(t + b + 2 < _GPW)
            def _():
                for c in in_copy(g + 2, b):
                    c.start()

    for b in (0, 1):
        for c in out_copy(g0 + _GPW - 2 + b, b):
            c.wait()


@jax.jit
def kernel(x, y, z):
    sx, sy, sz = (s["slab"] for s in _SPECS)
    ox, oy, oz = (s["olen"] for s in _SPECS)
    run = pl.kernel(
        _sc_fold_kernel,
        out_type=(
            jax.ShapeDtypeStruct((_ROWS * ox,), jnp.float32),
            jax.ShapeDtypeStruct((_ROWS * oy,), jnp.float32),
            jax.ShapeDtypeStruct((_ROWS * oz,), jnp.float32),
        ),
        mesh=plsc.VectorSubcoreMesh(core_axis_name="c", subcore_axis_name="s"),
        scratch_types=(
            pltpu.VMEM((_B * sx + _PAD,), jnp.float32),
            pltpu.VMEM((_B * sy + _PAD,), jnp.float32),
            pltpu.VMEM((_B * sz + _PAD,), jnp.float32),
            pltpu.VMEM((_B * sx + _PAD,), jnp.float32),
            pltpu.VMEM((_B * sy + _PAD,), jnp.float32),
            pltpu.VMEM((_B * sz + _PAD,), jnp.float32),
            pltpu.VMEM((_B * ox + _PAD,), jnp.float32),
            pltpu.VMEM((_B * oy + _PAD,), jnp.float32),
            pltpu.VMEM((_B * oz + _PAD,), jnp.float32),
            pltpu.VMEM((_B * ox + _PAD,), jnp.float32),
            pltpu.VMEM((_B * oy + _PAD,), jnp.float32),
            pltpu.VMEM((_B * oz + _PAD,), jnp.float32),
            pltpu.SemaphoreType.DMA,
            pltpu.SemaphoreType.DMA,
            pltpu.SemaphoreType.DMA,
            pltpu.SemaphoreType.DMA,
        ),
    )
    xo, yo, zo = run(x.reshape(-1), y.reshape(-1), z.reshape(-1))
    return (xo.reshape(64, 128, 22, 22),
            yo.reshape(64, 128, 17, 18),
            zo.reshape(64, 128, 5, 11))


# speedup vs baseline: 1.5811x; 1.5422x over previous
import jax
import jax.numpy as jnp
from jax import lax
from jax.experimental import pallas as pl
from jax.experimental.pallas import tpu as pltpu
from jax.experimental.pallas import tpu_sc as plsc

_LANES = 16
_NC, _NS = 2, 16
_NW = _NC * _NS
_ROWS = 64 * 128
_B = 8
_GROUPS = _ROWS // _B
_GPW = _GROUPS // _NW
_HDR = 16


def _fold_spec(oh, ow, kh, kw, sh, sw, ph, pw, dh, dw):
    assert sw == 1, "all three folds have unit output-width stride"
    Lh = (oh + 2 * ph - dh * (kh - 1) - 1) // sh + 1
    Lw = (ow + 2 * pw - dw * (kw - 1) - 1) // sw + 1
    slab = kh * kw * Lh * Lw
    olen = oh * ow
    rows = [[] for _ in range(oh)]
    for ki in range(kh):
        for kj in range(kw):
            for lh in range(Lh):
                r = lh * sh + ki * dh - ph
                if r < 0 or r >= oh:
                    continue
                c0 = kj * dw - pw
                s = max(0, c0)
                e = min(Lw + c0, ow)
                if e <= s:
                    continue
                rows[r].append((((ki * kw + kj) * Lh + lh) * Lw - c0, s, e))
    vecs = []
    for r in range(oh):
        for k in range(0, ow, _LANES):
            contribs = []
            for src0, s, e in rows[r]:
                a = max(s - k, 0)
                b = min(e - k, _LANES)
                if b > a:
                    contribs.append((src0 + k, a, b))
            assert contribs
            vecs.append((r * ow + k, contribs))
    tail = olen - vecs[-1][0]
    return dict(slab=slab, olen=olen, tail=tail, vecs=vecs)


_SPECS = (
    _fold_spec(22, 22, 3, 3, 1, 1, 0, 0, 1, 1),
    _fold_spec(17, 18, 2, 4, 2, 1, 2, 2, 1, 1),
    _fold_spec(5, 11, 2, 3, 1, 1, 2, 4, 1, 2),
)


def _sc_fold_kernel(xh, yh, zh, oxh, oyh, ozh,
                    ix0, iy0, iz0, ix1, iy1, iz1,
                    ox0, oy0, oz0, ox1, oy1, oz1,
                    si0, si1, so0, so1):
    wid = lax.axis_index("s") * _NC + lax.axis_index("c")
    g0 = wid * _GPW
    ins = (xh, yh, zh)
    outs = (oxh, oyh, ozh)
    in_slots = ((ix0, iy0, iz0), (ix1, iy1, iz1))
    out_slots = ((ox0, oy0, oz0), (ox1, oy1, oz1))
    in_sems = (si0, si1)
    out_sems = (so0, so1)

    iota = lax.iota(jnp.int32, _LANES)
    mask_keys = sorted({(a, b)
                        for spec in _SPECS
                        for _, contribs in spec["vecs"]
                        for (_, a, b) in contribs if (a, b) != (0, _LANES)}
                       | {(0, spec["tail"]) for spec in _SPECS})
    masks = {ab: (iota >= ab[0]) & (iota < ab[1]) for ab in mask_keys}

    def in_copy(g, slot):
        for hbm, buf, spec in zip(ins, in_slots[slot], _SPECS):
            sz = _B * spec["slab"]
            yield pltpu.make_async_copy(
                hbm.at[pl.ds(g * sz, sz)],
                buf.at[pl.ds(_HDR, sz)], in_sems[slot])

    def out_copy(g, slot):
        for hbm, buf, spec in zip(outs, out_slots[slot], _SPECS):
            sz = _B * spec["olen"]
            yield pltpu.make_async_copy(
                buf.at[pl.ds(0, sz)],
                hbm.at[pl.ds(g * sz, sz)], out_sems[slot])

    def compute_group(slot):
        bufs_i = in_slots[slot]
        bufs_o = out_slots[slot]

        @pl.loop(0, _B)
        def _(i):
            for buf_i, buf_o, spec in zip(bufs_i, bufs_o, _SPECS):
                base = _HDR + i * spec["slab"]
                obase = i * spec["olen"]
                last = spec["vecs"][-1][0]
                pend = []

                def flush(pend):
                    for o2, a2 in pend:
                        buf_o[pl.ds(obase + o2, _LANES)] = a2

                for off, contribs in spec["vecs"]:
                    acc = None
                    for lo, a, b in contribs:
                        v = buf_i[pl.ds(base + lo, _LANES)]
                        if (a, b) != (0, _LANES):
                            v = jnp.where(masks[(a, b)], v, 0.0)
                        acc = v if acc is None else acc + v
                    pend.append((off, acc))
                    if len(pend) == 4:
                        flush(pend)
                        pend = []
                flush(pend)

    for c in in_copy(g0, 0):
        c.start()
    for c in in_copy(g0 + 1, 1):
        c.start()

    @pl.loop(0, _GPW, step=2)
    def _(t):
        for slot in (0, 1):
            g = g0 + t + slot
            for c in in_copy(g, slot):
                c.wait()

            @pl.when(t + slot >= 2)
            def _():
                for c in out_copy(g - 2, slot):
                    c.wait()

            compute_group(slot)
            for c in out_copy(g, slot):
                c.start()

            @pl.when(t + slot + 2 < _GPW)
            def _():
                for c in in_copy(g + 2, slot):
                    c.start()

    for slot in (0, 1):
        for c in out_copy(g0 + _GPW - 2 + slot, slot):
            c.wait()


@jax.jit
def kernel(x, y, z):
    sx, sy, sz = (s["slab"] for s in _SPECS)
    run = pl.kernel(
        _sc_fold_kernel,
        out_type=tuple(
            jax.ShapeDtypeStruct((_ROWS * s["olen"],), jnp.float32)
            for s in _SPECS),
        mesh=plsc.VectorSubcoreMesh(core_axis_name="c", subcore_axis_name="s"),
        scratch_types=(
            pltpu.VMEM((_B * sx + 2 * _HDR,), jnp.float32),
            pltpu.VMEM((_B * sy + 2 * _HDR,), jnp.float32),
            pltpu.VMEM((_B * sz + 2 * _HDR,), jnp.float32),
            pltpu.VMEM((_B * sx + 2 * _HDR,), jnp.float32),
            pltpu.VMEM((_B * sy + 2 * _HDR,), jnp.float32),
            pltpu.VMEM((_B * sz + 2 * _HDR,), jnp.float32),
            pltpu.VMEM((_B * _SPECS[0]["olen"] + _HDR,), jnp.float32),
            pltpu.VMEM((_B * _SPECS[1]["olen"] + _HDR,), jnp.float32),
            pltpu.VMEM((_B * _SPECS[2]["olen"] + _HDR,), jnp.float32),
            pltpu.VMEM((_B * _SPECS[0]["olen"] + _HDR,), jnp.float32),
            pltpu.VMEM((_B * _SPECS[1]["olen"] + _HDR,), jnp.float32),
            pltpu.VMEM((_B * _SPECS[2]["olen"] + _HDR,), jnp.float32),
            pltpu.SemaphoreType.DMA,
            pltpu.SemaphoreType.DMA,
            pltpu.SemaphoreType.DMA,
            pltpu.SemaphoreType.DMA,
        ),
    )
    xo, yo, zo = run(x.reshape(-1), y.reshape(-1), z.reshape(-1))
    return (xo.reshape(64, 128, 22, 22),
            yo.reshape(64, 128, 17, 18),
            zo.reshape(64, 128, 5, 11))


# speedup vs baseline: 1.7568x; 1.1111x over previous
import jax
import jax.numpy as jnp
from jax import lax
from jax.experimental import pallas as pl
from jax.experimental.pallas import tpu as pltpu
from jax.experimental.pallas import tpu_sc as plsc

_LANES = 16
_NC, _NS = 2, 16
_NW = _NC * _NS
_ROWS = 64 * 128
_B = 8
_GROUPS = _ROWS // _B
_GPW = _GROUPS // _NW
_SLOT = 48


def _fold_spec(oh, ow, kh, kw, sh, sw, ph, pw, dh, dw, flat_out):
    assert sw == 1
    Lh = (oh + 2 * ph - dh * (kh - 1) - 1) // sh + 1
    Lw = (ow + 2 * pw - dw * (kw - 1) - 1) // sw + 1
    L = Lh * Lw
    ntap = kh * kw
    rows = [[] for _ in range(oh)]
    for ki in range(kh):
        for kj in range(kw):
            for lh in range(Lh):
                r = lh * sh + ki * dh - ph
                if r < 0 or r >= oh:
                    continue
                c0 = kj * dw - pw
                s = max(0, c0)
                e = min(Lw + c0, ow)
                if e > s:
                    rows[r].append((ki * kw + kj, lh, s, e, c0))
    ks = (0,) if ow <= _LANES else (0, ow - _LANES)
    side = {}
    vecs = []
    for r in range(oh):
        for k in ks:
            contribs = []
            for tap, lh, s, e, c0 in rows[r]:
                a = max(s - k, 0)
                b = min(e - k, _LANES)
                if b <= a:
                    continue
                col = lh * Lw + k - c0
                if col < 0 or col + _LANES > L:
                    base = 0 if col < 0 else L - _LANES
                    slot = side.setdefault((tap, base), len(side))
                    contribs.append((None, slot * _SLOT + 16 + col - base, a, b))
                else:
                    contribs.append((tap, col, a, b))
            assert contribs
            vecs.append((r, k, contribs))
    return dict(L=L, ntap=ntap, oh=oh, ow=ow, olen=oh * ow, vecs=vecs,
                side=sorted(side.items(), key=lambda kv: kv[1]),
                flat_out=flat_out)


_SPECS = (
    _fold_spec(22, 22, 3, 3, 1, 1, 0, 0, 1, 1, False),
    _fold_spec(17, 18, 2, 4, 2, 1, 2, 2, 1, 1, False),
    _fold_spec(5, 11, 2, 3, 1, 1, 2, 4, 1, 2, True),
)


def _sc_fold_kernel(xh, yh, zh, oxh, oyh, ozh,
                    ix0, iy0, ix1, iy1, izb,
                    obx, oby, obz, sbx, sby, sbz,
                    si0, si1, siz, so):
    wid = lax.axis_index("s") * _NC + lax.axis_index("c")
    g0 = wid * _GPW
    in_slots = ((ix0, iy0), (ix1, iy1))
    in_sems = (si0, si1)
    obufs = (obx, oby, obz)
    sbufs = (sbx, sby, sbz)

    iota = lax.iota(jnp.int32, _LANES)
    mask_keys = sorted({(a, b)
                        for spec in _SPECS
                        for _, _, contribs in spec["vecs"]
                        for (_, _, a, b) in contribs if (a, b) != (0, _LANES)})
    masks = {ab: (iota >= ab[0]) & (iota < ab[1]) for ab in mask_keys}

    def in_copy(g, slot):
        for hbm, buf, spec in zip((xh, yh), in_slots[slot], _SPECS[:2]):
            nr = _B * spec["ntap"]
            yield pltpu.make_async_copy(
                hbm.at[pl.ds(g * nr, nr), :], buf, in_sems[slot])

    def z_in_copy(g):
        nr = _B * _SPECS[2]["ntap"]
        return pltpu.make_async_copy(zh.at[pl.ds(g * nr, nr), :], izb, siz)

    def wave_copies(row0):
        for hbm, buf in ((oxh, obx), (oyh, oby)):
            yield pltpu.make_async_copy(
                buf, hbm.at[pl.ds(row0, 2), :, :], so)

    def z_out_copy(g):
        sz = _B * _SPECS[2]["olen"]
        return pltpu.make_async_copy(
            obz.at[pl.ds(0, sz)], ozh.at[pl.ds(g * sz, sz)], so)

    def compute_sample(bufs_i, i):
        for buf_i, buf_o, sbuf, spec in zip(bufs_i, obufs, sbufs, _SPECS):
            ntap = spec["ntap"]
            for (tap, base), slot_i in spec["side"]:
                sbuf[pl.ds(slot_i * _SLOT + 16, _LANES)] = (
                    buf_i[i * ntap + tap, pl.ds(base, _LANES)])
            pend = []

            def flush(pend):
                for r, k, acc in pend:
                    if spec["flat_out"]:
                        buf_o[pl.ds(i * spec["olen"] + r * spec["ow"],
                                    _LANES)] = acc
                    else:
                        buf_o[i % 2, r, pl.ds(k, _LANES)] = acc

            for r, k, contribs in spec["vecs"]:
                acc = None
                for tap, col, a, b in contribs:
                    if tap is None:
                        v = sbuf[pl.ds(col, _LANES)]
                    else:
                        v = buf_i[i * ntap + tap, pl.ds(col, _LANES)]
                    if (a, b) != (0, _LANES):
                        v = jnp.where(masks[(a, b)], v, 0.0)
                    acc = v if acc is None else acc + v
                pend.append((r, k, acc))
                if len(pend) == 4:
                    flush(pend)
                    pend = []
            flush(pend)

    for c in in_copy(g0, 0):
        c.start()
    for c in in_copy(g0 + 1, 1):
        c.start()
    z_in_copy(g0).start()

    @pl.loop(0, _GPW, step=2)
    def _(t):
        for slot in (0, 1):
            g = g0 + t + slot
            for c in in_copy(g, slot):
                c.wait()
            z_in_copy(g).wait()
            bufs_i = (in_slots[slot][0], in_slots[slot][1], izb)

            @pl.loop(0, _B)
            def _(i):
                @pl.when((i % 2 == 0) & ((i > 0) | (g > g0)))
                def _():
                    for c in wave_copies(0):
                        c.wait()

                @pl.when((i == 0) & (g > g0))
                def _():
                    z_out_copy(0).wait()

                compute_sample(bufs_i, i)

                @pl.when(i % 2 == 1)
                def _():
                    for c in wave_copies(g * _B + i - 1):
                        c.start()

                @pl.when(i == _B - 1)
                def _():
                    z_out_copy(g).start()

            @pl.when(g + 1 < g0 + _GPW)
            def _():
                z_in_copy(g + 1).start()

            @pl.when(t + slot + 2 < _GPW)
            def _():
                for c in in_copy(g + 2, slot):
                    c.start()

    for c in wave_copies(0):
        c.wait()
    z_out_copy(0).wait()


@jax.jit
def kernel(x, y, z):
    x2 = x.reshape(64 * 1152, 400)
    y2 = y.reshape(64 * 1024, 190)
    z2 = z.reshape(64 * 768, 120)
    run = pl.kernel(
        _sc_fold_kernel,
        out_type=(
            jax.ShapeDtypeStruct((_ROWS, 22, 22), jnp.float32),
            jax.ShapeDtypeStruct((_ROWS, 17, 18), jnp.float32),
            jax.ShapeDtypeStruct((_ROWS * 55,), jnp.float32),
        ),
        mesh=plsc.VectorSubcoreMesh(core_axis_name="c", subcore_axis_name="s"),
        scratch_types=(
            pltpu.VMEM((_B * 9, 400), jnp.float32),
            pltpu.VMEM((_B * 8, 190), jnp.float32),
            pltpu.VMEM((_B * 9, 400), jnp.float32),
            pltpu.VMEM((_B * 8, 190), jnp.float32),
            pltpu.VMEM((_B * 6, 120), jnp.float32),
            pltpu.VMEM((2, 22, 22), jnp.float32),
            pltpu.VMEM((2, 17, 18), jnp.float32),
            pltpu.VMEM((_B * 55 + 16,), jnp.float32),
            pltpu.VMEM((max(len(_SPECS[0]["side"]), 1) * _SLOT,), jnp.float32),
            pltpu.VMEM((max(len(_SPECS[1]["side"]), 1) * _SLOT,), jnp.float32),
            pltpu.VMEM((max(len(_SPECS[2]["side"]), 1) * _SLOT,), jnp.float32),
            pltpu.SemaphoreType.DMA,
            pltpu.SemaphoreType.DMA,
            pltpu.SemaphoreType.DMA,
            pltpu.SemaphoreType.DMA,
        ),
    )
    xo, yo, zo = run(x2, y2, z2)
    return (xo.reshape(64, 128, 22, 22),
            yo.reshape(64, 128, 17, 18),
            zo.reshape(64, 128, 5, 11))
